# Initial kernel scaffold; baseline (speedup 1.0000x reference)
import jax
import jax.numpy as jnp
from jax.experimental import pallas as pl


def kernel(x, W1a, b1a, W2a, b2a, W1b, b1b, W2b, b2b, Wm1, bm1, Wm2, bm2, edge_index, batch):
    raise NotImplementedError("write your pallas kernel here")


# speedup vs baseline: 1.8408x; 1.8408x over previous
import functools

import jax
import jax.numpy as jnp
from jax import lax
from jax.experimental import pallas as pl
from jax.experimental.pallas import tpu as pltpu
from jax.experimental.pallas import tpu_sc as plsc

NN = 100000
EE = 1600000
FF = 16
HH = 64
GG = 16

NSC = 2
NTILE = 16
BK = 25000
NBUCKET = 4
SP_ROWS = 25600
ROWS_PER_TILE = 1600
TRASH = BK
CH = 128
NCHUNK = EE // CH
CHUNK_ITERS = (NCHUNK + NTILE - 1) // NTILE
ZR = 160

RB = 2000
NROWB = NN // RB

_HI = jax.lax.Precision.HIGHEST


def _dot(a, b):
    return jnp.dot(a, b, precision=_HI, preferred_element_type=jnp.float32)


def _edge_body(A_hbm, B_hbm, src_hbm, dst_hbm, S_out,
               srcv, dstv, lidxv, av, bv, zb, S_sp, sem1, sem2):
    c = lax.axis_index("c")
    s = lax.axis_index("s")

    def _zinit(e, _):
        for g in range(4):
            zb[e, pl.ds(g * 16, 16)] = jnp.zeros((16,), jnp.float32)
        return 0
    lax.fori_loop(0, ZR, _zinit, 0)

    for p in range(2):
        b = c * 2 + p
        lo = b * BK
        base = s * ROWS_PER_TILE

        for j in range(ROWS_PER_TILE // ZR):
            pltpu.sync_copy(zb, S_sp.at[pl.ds(base + j * ZR, ZR)])
        plsc.subcore_barrier()

        def _chunk(i, _):
            cid = i * NTILE + s

            @pl.when(cid < NCHUNK)
            def _():
                off = cid * CH
                pltpu.sync_copy(src_hbm.at[pl.ds(off, CH)], srcv)
                pltpu.sync_copy(dst_hbm.at[pl.ds(off, CH)], dstv)
                for g in range(CH // 16):
                    d = dstv[pl.ds(g * 16, 16)]
                    l = d - lo
                    ok = (l >= 0) & (l < BK)
                    lidxv[pl.ds(g * 16, 16)] = jnp.where(ok, l, TRASH)
                cp1 = pltpu.async_copy(A_hbm.at[dstv], av, sem1)
                cp2 = pltpu.async_copy(B_hbm.at[srcv], bv, sem2)
                cp1.wait()
                cp2.wait()

                def _relu(e, _):
                    for g in range(HH // 16):
                        sl = pl.ds(g * 16, 16)
                        av[e, sl] = jnp.maximum(av[e, sl] + bv[e, sl], 0.0)
                    return 0
                lax.fori_loop(0, CH, _relu, 0)

                pltpu.sync_copy(av, S_sp.at[lidxv], add=True)
            return 0
        lax.fori_loop(0, CHUNK_ITERS, _chunk, 0)
        plsc.subcore_barrier()

        pltpu.sync_copy(S_sp.at[pl.ds(base, ROWS_PER_TILE)],
                        S_out.at[b, pl.ds(base, ROWS_PER_TILE)])
        plsc.subcore_barrier()


def _deg_body(dst_hbm, deg_out, dstv, lidxv, onesv, zbd, deg_sp, sem1):
    c = lax.axis_index("c")
    s = lax.axis_index("s")

    def _zinit(e, _):
        zbd[e, :] = jnp.zeros((16,), jnp.float32)
        return 0
    lax.fori_loop(0, ZR, _zinit, 0)

    def _oinit(e, _):
        onesv[e, :] = jnp.ones((16,), jnp.float32)
        return 0
    lax.fori_loop(0, CH, _oinit, 0)

    for p in range(2):
        b = c * 2 + p
        lo = b * BK
        base = s * ROWS_PER_TILE
        for j in range(ROWS_PER_TILE // ZR):
            pltpu.sync_copy(zbd, deg_sp.at[pl.ds(base + j * ZR, ZR)])
        plsc.subcore_barrier()

        def _chunk(i, _):
            cid = i * NTILE + s

            @pl.when(cid < NCHUNK)
            def _():
                off = cid * CH
                pltpu.sync_copy(dst_hbm.at[pl.ds(off, CH)], dstv)
                for g in range(CH // 16):
                    d = dstv[pl.ds(g * 16, 16)]
                    l = d - lo
                    ok = (l >= 0) & (l < BK)
                    lidxv[pl.ds(g * 16, 16)] = jnp.where(ok, l, TRASH)
                pltpu.sync_copy(onesv, deg_sp.at[lidxv], add=True)
            return 0
        lax.fori_loop(0, CHUNK_ITERS, _chunk, 0)
        plsc.subcore_barrier()

        pltpu.sync_copy(deg_sp.at[pl.ds(base, ROWS_PER_TILE)],
                        deg_out.at[b, pl.ds(base, ROWS_PER_TILE)])
        plsc.subcore_barrier()


def _make_edge_kernel():
    mesh = plsc.VectorSubcoreMesh(core_axis_name="c", subcore_axis_name="s")
    scratch = [
        pltpu.VMEM((CH,), jnp.int32),
        pltpu.VMEM((CH,), jnp.int32),
        pltpu.VMEM((CH,), jnp.int32),
        pltpu.VMEM((CH, HH), jnp.float32),
        pltpu.VMEM((CH, HH), jnp.float32),
        pltpu.VMEM((ZR, HH), jnp.float32),
        pltpu.VMEM_SHARED((SP_ROWS, HH), jnp.float32),
        pltpu.SemaphoreType.DMA,
        pltpu.SemaphoreType.DMA,
    ]
    return pl.kernel(_edge_body,
                     out_type=jax.ShapeDtypeStruct((NBUCKET, SP_ROWS, HH),
                                                   jnp.float32),
                     mesh=mesh, scratch_types=scratch,
                     compiler_params=pltpu.CompilerParams(
                         use_tc_tiling_on_sc=False))


def _make_deg_kernel():
    mesh = plsc.VectorSubcoreMesh(core_axis_name="c", subcore_axis_name="s")
    scratch = [
        pltpu.VMEM((CH,), jnp.int32),
        pltpu.VMEM((CH,), jnp.int32),
        pltpu.VMEM((CH, 16), jnp.float32),
        pltpu.VMEM((ZR, 16), jnp.float32),
        pltpu.VMEM_SHARED((SP_ROWS, 16), jnp.float32),
        pltpu.SemaphoreType.DMA,
    ]
    return pl.kernel(_deg_body,
                     out_type=jax.ShapeDtypeStruct((NBUCKET, SP_ROWS, 16),
                                                   jnp.float32),
                     mesh=mesh, scratch_types=scratch,
                     compiler_params=pltpu.CompilerParams(
                         use_tc_tiling_on_sc=False))


_edge_kernel = _make_edge_kernel()
_deg_kernel = _make_deg_kernel()


def _pre_body(x_ref, wt_ref, wb_ref, b1_ref, a_ref, bo_ref):
    xb = x_ref[...]
    a_ref[...] = _dot(xb, wt_ref[...]) + b1_ref[...]
    bo_ref[...] = _dot(xb, wb_ref[...])


def _tc_pre(x, wt, wb, b1row):
    return pl.pallas_call(
        _pre_body,
        grid=(NROWB,),
        in_specs=[
            pl.BlockSpec((RB, FF), lambda i: (i, 0)),
            pl.BlockSpec((FF, HH), lambda i: (0, 0)),
            pl.BlockSpec((FF, HH), lambda i: (0, 0)),
            pl.BlockSpec((1, HH), lambda i: (0, 0)),
        ],
        out_specs=[pl.BlockSpec((RB, HH), lambda i: (i, 0))] * 2,
        out_shape=[jax.ShapeDtypeStruct((NN, HH), jnp.float32)] * 2,
    )(x, wt, wb, b1row)


def _mid_body(s_ref, deg_ref, w2_ref, b2_ref, wt_ref, wb_ref, b1_ref,
              a_ref, bo_ref):
    dcol = deg_ref[:, 0:1]
    h = jnp.maximum(_dot(s_ref[...], w2_ref[...]) + dcol * b2_ref[...], 0.0)
    a_ref[...] = _dot(h, wt_ref[...]) + b1_ref[...]
    bo_ref[...] = _dot(h, wb_ref[...])


def _tc_mid(S1, deg16, w2, b2row, wt, wb, b1row):
    return pl.pallas_call(
        _mid_body,
        grid=(NROWB,),
        in_specs=[
            pl.BlockSpec((RB, HH), lambda i: (i, 0)),
            pl.BlockSpec((RB, 16), lambda i: (i, 0)),
            pl.BlockSpec((HH, HH), lambda i: (0, 0)),
            pl.BlockSpec((1, HH), lambda i: (0, 0)),
            pl.BlockSpec((HH, HH), lambda i: (0, 0)),
            pl.BlockSpec((HH, HH), lambda i: (0, 0)),
            pl.BlockSpec((1, HH), lambda i: (0, 0)),
        ],
        out_specs=[pl.BlockSpec((RB, HH), lambda i: (i, 0))] * 2,
        out_shape=[jax.ShapeDtypeStruct((NN, HH), jnp.float32)] * 2,
    )(S1, deg16, w2, b2row, wt, wb, b1row)


def _post_body(s_ref, deg_ref, batch_ref, w2_ref, b2_ref,
               wm1_ref, bm1_ref, wm2_ref, bm2_ref,
               out_ref, sums, cnts):
    i = pl.program_id(0)

    @pl.when(i == 0)
    def _():
        sums[...] = jnp.zeros_like(sums)
        cnts[...] = jnp.zeros_like(cnts)

    dcol = deg_ref[:, 0:1]
    h2 = jnp.maximum(_dot(s_ref[...], w2_ref[...]) + dcol * b2_ref[...], 0.0)
    bb = batch_ref[0]
    gids = lax.broadcasted_iota(jnp.int32, (GG, RB), 0)
    oh = (bb == gids).astype(jnp.float32)
    sums[...] += lax.dot_general(oh, h2, (((1,), (0,)), ((), ())),
                                 precision=_HI,
                                 preferred_element_type=jnp.float32)
    cnts[...] += lax.dot_general(oh, jnp.ones((RB, HH), jnp.float32),
                                 (((1,), (0,)), ((), ())),
                                 precision=_HI,
                                 preferred_element_type=jnp.float32)

    @pl.when(i == NROWB - 1)
    def _():
        pooled = sums[...] / jnp.maximum(cnts[...], 1.0)
        lat = jnp.maximum(_dot(pooled, wm1_ref[...]) + bm1_ref[...], 0.0)
        out_ref[...] = _dot(lat, wm2_ref[...]) + bm2_ref[...]


def _tc_post(S2, deg16, batch3, w2, b2row, wm1, bm1row, wm2, bm2row):
    return pl.pallas_call(
        _post_body,
        grid=(NROWB,),
        in_specs=[
            pl.BlockSpec((RB, HH), lambda i: (i, 0)),
            pl.BlockSpec((RB, 16), lambda i: (i, 0)),
            pl.BlockSpec((1, 1, RB), lambda i: (i, 0, 0)),
            pl.BlockSpec((HH, HH), lambda i: (0, 0)),
            pl.BlockSpec((1, HH), lambda i: (0, 0)),
            pl.BlockSpec((HH, HH), lambda i: (0, 0)),
            pl.BlockSpec((1, HH), lambda i: (0, 0)),
            pl.BlockSpec((HH, 2 * HH), lambda i: (0, 0)),
            pl.BlockSpec((1, 2 * HH), lambda i: (0, 0)),
        ],
        out_specs=pl.BlockSpec((GG, 2 * HH), lambda i: (0, 0)),
        out_shape=jax.ShapeDtypeStruct((GG, 2 * HH), jnp.float32),
        scratch_shapes=[
            pltpu.VMEM((GG, HH), jnp.float32),
            pltpu.VMEM((GG, HH), jnp.float32),
        ],
    )(S2, deg16, batch3, w2, b2row, wm1, bm1row, wm2, bm2row)


def kernel(x, W1a, b1a, W2a, b2a, W1b, b1b, W2b, b2b, Wm1, bm1, Wm2, bm2,
           edge_index, batch):
    src = edge_index[0].astype(jnp.int32)
    dst = edge_index[1].astype(jnp.int32)

    A1, B1 = _tc_pre(x, W1a[:FF], W1a[FF:], b1a[None, :])
    deg4 = _deg_kernel(dst)
    S1_4 = _edge_kernel(A1, B1, src, dst)
    S1 = S1_4[:, :BK, :].reshape(NN, HH)
    deg16 = deg4[:, :BK, :].reshape(NN, 16)

    A2, B2 = _tc_mid(S1, deg16, W2a, b2a[None, :], W1b[:HH], W1b[HH:],
                     b1b[None, :])
    S2_4 = _edge_kernel(A2, B2, src, dst)
    S2 = S2_4[:, :BK, :].reshape(NN, HH)

    lat = _tc_post(S2, deg16, batch.astype(jnp.int32).reshape(NROWB, 1, RB),
                   W2b, b2b[None, :], Wm1, bm1[None, :], Wm2, bm2[None, :])
    return lat.reshape(-1, 8, 16)


# speedup vs baseline: 3.1523x; 1.7125x over previous
import functools

import jax
import jax.numpy as jnp
from jax import lax
from jax.experimental import pallas as pl
from jax.experimental.pallas import tpu as pltpu
from jax.experimental.pallas import tpu_sc as plsc

NN = 100000
EE = 1600000
FF = 16
HH = 64
GG = 16

NSC = 2
NTILE = 16
NGRP = 4
GW = 16
RPT = NN // NTILE
CH = 128
NCHUNK = EE // CH
CHUNK_ITERS = (NCHUNK + NTILE - 1) // NTILE
HCHUNK = NCHUNK // NSC
HITERS = (HCHUNK + NTILE - 1) // NTILE
ZR = 250

RB = 2000
NROWB = NN // RB

_HI = jax.lax.Precision.HIGHEST


def _dot(a, b):
    return jnp.dot(a, b, precision=_HI, preferred_element_type=jnp.float32)


def _edge_body(A_hbm, B_hbm, src_hbm, dst_hbm, S_out,
               srcv, dstv, gidxv, av, bv, zb, S_sp, sem1, sem2):
    c = lax.axis_index("c")
    s = lax.axis_index("s")

    def _zinit(e, _):
        zb[e, :] = jnp.zeros((GW,), jnp.float32)
        return 0
    lax.fori_loop(0, ZR, _zinit, 0)

    base = s * RPT
    for p in range(2):
        g = c * 2 + p
        off_g = g * NN

        for j in range(RPT // ZR):
            pltpu.sync_copy(zb, S_sp.at[pl.ds(base + j * ZR, ZR)])
        plsc.subcore_barrier()

        def _chunk(i, _):
            cid = i * NTILE + s

            @pl.when(cid < NCHUNK)
            def _():
                off = cid * CH
                pltpu.sync_copy(src_hbm.at[pl.ds(off, CH)], srcv)
                pltpu.sync_copy(dst_hbm.at[pl.ds(off, CH)], dstv)
                for q in range(CH // 16):
                    sl = pl.ds(q * 16, 16)
                    gidxv[sl] = dstv[sl] + off_g
                    srcv[sl] = srcv[sl] + off_g
                cp1 = pltpu.async_copy(A_hbm.at[gidxv], av, sem1)
                cp2 = pltpu.async_copy(B_hbm.at[srcv], bv, sem2)
                cp1.wait()
                cp2.wait()

                def _relu(e, _):
                    rb = e * 8
                    for u in range(8):
                        av[rb + u, :] = jnp.maximum(
                            av[rb + u, :] + bv[rb + u, :], 0.0)
                    return 0
                lax.fori_loop(0, CH // 8, _relu, 0)

                pltpu.sync_copy(av, S_sp.at[dstv], add=True)
            return 0
        lax.fori_loop(0, CHUNK_ITERS, _chunk, 0)
        plsc.subcore_barrier()

        pltpu.sync_copy(S_sp.at[pl.ds(base, RPT)],
                        S_out.at[g, pl.ds(base, RPT)])
        plsc.subcore_barrier()


def _deg_body(dst_hbm, deg_out, dstv, onesv, zbd, deg_sp, sem1):
    c = lax.axis_index("c")
    s = lax.axis_index("s")

    def _zinit(e, _):
        zbd[e, :] = jnp.zeros((GW,), jnp.float32)
        return 0
    lax.fori_loop(0, ZR, _zinit, 0)

    def _oinit(e, _):
        onesv[e, :] = jnp.ones((GW,), jnp.float32)
        return 0
    lax.fori_loop(0, CH, _oinit, 0)

    base = s * RPT
    for j in range(RPT // ZR):
        pltpu.sync_copy(zbd, deg_sp.at[pl.ds(base + j * ZR, ZR)])
    plsc.subcore_barrier()

    def _chunk(i, _):
        k = i * NTILE + s

        @pl.when(k < HCHUNK)
        def _():
            off = (c * HCHUNK + k) * CH
            pltpu.sync_copy(dst_hbm.at[pl.ds(off, CH)], dstv)
            pltpu.sync_copy(onesv, deg_sp.at[dstv], add=True)
        return 0
    lax.fori_loop(0, HITERS, _chunk, 0)
    plsc.subcore_barrier()

    pltpu.sync_copy(deg_sp.at[pl.ds(base, RPT)],
                    deg_out.at[c, pl.ds(base, RPT)])
    plsc.subcore_barrier()


def _make_edge_kernel():
    mesh = plsc.VectorSubcoreMesh(core_axis_name="c", subcore_axis_name="s")
    scratch = [
        pltpu.VMEM((CH,), jnp.int32),
        pltpu.VMEM((CH,), jnp.int32),
        pltpu.VMEM((CH,), jnp.int32),
        pltpu.VMEM((CH, GW), jnp.float32),
        pltpu.VMEM((CH, GW), jnp.float32),
        pltpu.VMEM((ZR, GW), jnp.float32),
        pltpu.VMEM_SHARED((NN, GW), jnp.float32),
        pltpu.SemaphoreType.DMA,
        pltpu.SemaphoreType.DMA,
    ]
    return pl.kernel(_edge_body,
                     out_type=jax.ShapeDtypeStruct((NGRP, NN, GW),
                                                   jnp.float32),
                     mesh=mesh, scratch_types=scratch,
                     compiler_params=pltpu.CompilerParams(
                         use_tc_tiling_on_sc=False))


def _make_deg_kernel():
    mesh = plsc.VectorSubcoreMesh(core_axis_name="c", subcore_axis_name="s")
    scratch = [
        pltpu.VMEM((CH,), jnp.int32),
        pltpu.VMEM((CH, GW), jnp.float32),
        pltpu.VMEM((ZR, GW), jnp.float32),
        pltpu.VMEM_SHARED((NN, GW), jnp.float32),
        pltpu.SemaphoreType.DMA,
    ]
    return pl.kernel(_deg_body,
                     out_type=jax.ShapeDtypeStruct((NSC, NN, GW),
                                                   jnp.float32),
                     mesh=mesh, scratch_types=scratch,
                     compiler_params=pltpu.CompilerParams(
                         use_tc_tiling_on_sc=False))


_edge_kernel = _make_edge_kernel()
_deg_kernel = _make_deg_kernel()


def _split_groups(res, ref):
    for g in range(NGRP):
        ref[g] = res[:, g * GW:(g + 1) * GW]


def _pre_body(x_ref, wt_ref, wb_ref, b1_ref, a_ref, bo_ref):
    xb = x_ref[...]
    _split_groups(_dot(xb, wt_ref[...]) + b1_ref[...], a_ref)
    _split_groups(_dot(xb, wb_ref[...]), bo_ref)


def _tc_pre(x, wt, wb, b1row):
    return pl.pallas_call(
        _pre_body,
        grid=(NROWB,),
        in_specs=[
            pl.BlockSpec((RB, FF), lambda i: (i, 0)),
            pl.BlockSpec((FF, HH), lambda i: (0, 0)),
            pl.BlockSpec((FF, HH), lambda i: (0, 0)),
            pl.BlockSpec((1, HH), lambda i: (0, 0)),
        ],
        out_specs=[pl.BlockSpec((NGRP, RB, GW), lambda i: (0, i, 0))] * 2,
        out_shape=[jax.ShapeDtypeStruct((NGRP, NN, GW), jnp.float32)] * 2,
    )(x, wt, wb, b1row)


def _cat_groups(s_ref):
    return jnp.concatenate([s_ref[g] for g in range(NGRP)], axis=1)


def _mid_body(s_ref, deg_ref, w2_ref, b2_ref, wt_ref, wb_ref, b1_ref,
              a_ref, bo_ref):
    dcol = deg_ref[0][:, 0:1] + deg_ref[1][:, 0:1]
    h = jnp.maximum(_dot(_cat_groups(s_ref), w2_ref[...])
                    + dcol * b2_ref[...], 0.0)
    _split_groups(_dot(h, wt_ref[...]) + b1_ref[...], a_ref)
    _split_groups(_dot(h, wb_ref[...]), bo_ref)


def _tc_mid(S1, deg2, w2, b2row, wt, wb, b1row):
    return pl.pallas_call(
        _mid_body,
        grid=(NROWB,),
        in_specs=[
            pl.BlockSpec((NGRP, RB, GW), lambda i: (0, i, 0)),
            pl.BlockSpec((NSC, RB, GW), lambda i: (0, i, 0)),
            pl.BlockSpec((HH, HH), lambda i: (0, 0)),
            pl.BlockSpec((1, HH), lambda i: (0, 0)),
            pl.BlockSpec((HH, HH), lambda i: (0, 0)),
            pl.BlockSpec((HH, HH), lambda i: (0, 0)),
            pl.BlockSpec((1, HH), lambda i: (0, 0)),
        ],
        out_specs=[pl.BlockSpec((NGRP, RB, GW), lambda i: (0, i, 0))] * 2,
        out_shape=[jax.ShapeDtypeStruct((NGRP, NN, GW), jnp.float32)] * 2,
    )(S1, deg2, w2, b2row, wt, wb, b1row)


def _post_body(s_ref, deg_ref, batch_ref, w2_ref, b2_ref,
               wm1_ref, bm1_ref, wm2_ref, bm2_ref,
               out_ref, sums, cnts):
    i = pl.program_id(0)

    @pl.when(i == 0)
    def _():
        sums[...] = jnp.zeros_like(sums)
        cnts[...] = jnp.zeros_like(cnts)

    dcol = deg_ref[0][:, 0:1] + deg_ref[1][:, 0:1]
    h2 = jnp.maximum(_dot(_cat_groups(s_ref), w2_ref[...])
                     + dcol * b2_ref[...], 0.0)
    bb = batch_ref[0]
    gids = lax.broadcasted_iota(jnp.int32, (GG, RB), 0)
    oh = (bb == gids).astype(jnp.float32)
    sums[...] += lax.dot_general(oh, h2, (((1,), (0,)), ((), ())),
                                 precision=_HI,
                                 preferred_element_type=jnp.float32)
    cnts[...] += lax.dot_general(oh, jnp.ones((RB, HH), jnp.float32),
                                 (((1,), (0,)), ((), ())),
                                 precision=_HI,
                                 preferred_element_type=jnp.float32)

    @pl.when(i == NROWB - 1)
    def _():
        pooled = sums[...] / jnp.maximum(cnts[...], 1.0)
        lat = jnp.maximum(_dot(pooled, wm1_ref[...]) + bm1_ref[...], 0.0)
        out_ref[...] = _dot(lat, wm2_ref[...]) + bm2_ref[...]


def _tc_post(S2, deg2, batch3, w2, b2row, wm1, bm1row, wm2, bm2row):
    return pl.pallas_call(
        _post_body,
        grid=(NROWB,),
        in_specs=[
            pl.BlockSpec((NGRP, RB, GW), lambda i: (0, i, 0)),
            pl.BlockSpec((NSC, RB, GW), lambda i: (0, i, 0)),
            pl.BlockSpec((1, 1, RB), lambda i: (i, 0, 0)),
            pl.BlockSpec((HH, HH), lambda i: (0, 0)),
            pl.BlockSpec((1, HH), lambda i: (0, 0)),
            pl.BlockSpec((HH, HH), lambda i: (0, 0)),
            pl.BlockSpec((1, HH), lambda i: (0, 0)),
            pl.BlockSpec((HH, 2 * HH), lambda i: (0, 0)),
            pl.BlockSpec((1, 2 * HH), lambda i: (0, 0)),
        ],
        out_specs=pl.BlockSpec((GG, 2 * HH), lambda i: (0, 0)),
        out_shape=jax.ShapeDtypeStruct((GG, 2 * HH), jnp.float32),
        scratch_shapes=[
            pltpu.VMEM((GG, HH), jnp.float32),
            pltpu.VMEM((GG, HH), jnp.float32),
        ],
    )(S2, deg2, batch3, w2, b2row, wm1, bm1row, wm2, bm2row)


def kernel(x, W1a, b1a, W2a, b2a, W1b, b1b, W2b, b2b, Wm1, bm1, Wm2, bm2,
           edge_index, batch):
    src = edge_index[0].astype(jnp.int32)
    dst = edge_index[1].astype(jnp.int32)

    A1, B1 = _tc_pre(x, W1a[:FF], W1a[FF:], b1a[None, :])
    deg2 = _deg_kernel(dst)
    S1 = _edge_kernel(A1.reshape(NGRP * NN, GW), B1.reshape(NGRP * NN, GW),
                      src, dst)

    A2, B2 = _tc_mid(S1, deg2, W2a, b2a[None, :], W1b[:HH], W1b[HH:],
                     b1b[None, :])
    S2 = _edge_kernel(A2.reshape(NGRP * NN, GW), B2.reshape(NGRP * NN, GW),
                      src, dst)

    lat = _tc_post(S2, deg2, batch.astype(jnp.int32).reshape(NROWB, 1, RB),
                   W2b, b2b[None, :], Wm1, bm1[None, :], Wm2, bm2[None, :])
    return lat.reshape(-1, 8, 16)


# speedup vs baseline: 5.6289x; 1.7856x over previous
import functools

import jax
import jax.numpy as jnp
from jax import lax
from jax.experimental import pallas as pl
from jax.experimental.pallas import tpu as pltpu
from jax.experimental.pallas import tpu_sc as plsc

NN = 100000
EE = 1600000
FF = 16
HH = 64
GG = 16

NSC = 2
NTILE = 16
NGRP = 4
GW = 16
RPT = NN // NTILE
CH = 80
EPT = EE // NTILE
BLKE = 2000
SUBC = BLKE // CH
NBLK = EPT // BLKE
NCHUNK = EE // 128
HCHUNK = NCHUNK // NSC
HITERS = (HCHUNK + NTILE - 1) // NTILE
ZR = 250

RB = 2000
NROWB = NN // RB

_HI = jax.lax.Precision.HIGHEST


def _dot(a, b):
    return jnp.dot(a, b, precision=_HI, preferred_element_type=jnp.float32)


def _edge_body(A_hbm, B_hbm, src_hbm, dst_hbm, S_out,
               srcv, dstv, gidxv, av0, bv0, av1, bv1, zb, S_sp,
               sa0, sb0, sa1, sb1):
    c = lax.axis_index("c")
    s = lax.axis_index("s")

    def _zinit(e, _):
        zb[e, :] = jnp.zeros((GW,), jnp.float32)
        return 0
    lax.fori_loop(0, ZR, _zinit, 0)

    base = s * RPT
    ebase = s * EPT
    avs = [av0, av1]
    bvs = [bv0, bv1]
    sas = [sa0, sa1]
    sbs = [sb0, sb1]

    for p in range(2):
        off_g = (c * 2 + p) * NN

        for j in range(RPT // ZR):
            pltpu.sync_copy(zb, S_sp.at[pl.ds(base + j * ZR, ZR)])
        plsc.subcore_barrier()

        def _block(i, _):
            boff = ebase + i * BLKE
            pltpu.sync_copy(src_hbm.at[pl.ds(boff, BLKE)], srcv)
            pltpu.sync_copy(dst_hbm.at[pl.ds(boff, BLKE)], dstv)

            def _remap(e, _):
                sl = pl.ds(e * 16, 16)
                gidxv[sl] = dstv[sl] + off_g
                srcv[sl] = srcv[sl] + off_g
                return 0
            lax.fori_loop(0, BLKE // 16, _remap, 0)

            def _issue(j):
                sl = pl.ds(j * CH, CH)
                t = j % 2
                return (pltpu.async_copy(A_hbm.at[gidxv.at[sl]], avs[t],
                                         sas[t]),
                        pltpu.async_copy(B_hbm.at[srcv.at[sl]], bvs[t],
                                         sbs[t]))

            cur = _issue(0)
            for j in range(SUBC):
                t = j % 2
                nxt = _issue(j + 1) if j + 1 < SUBC else None
                cur[0].wait()
                cur[1].wait()
                av, bv = avs[t], bvs[t]

                def _relu(e, _):
                    rb = e * 8
                    for u in range(8):
                        av[rb + u, :] = jnp.maximum(
                            av[rb + u, :] + bv[rb + u, :], 0.0)
                    return 0
                lax.fori_loop(0, CH // 8, _relu, 0)

                pltpu.sync_copy(av, S_sp.at[dstv.at[pl.ds(j * CH, CH)]],
                                add=True)
                cur = nxt
            return 0
        lax.fori_loop(0, NBLK, _block, 0)
        plsc.subcore_barrier()

        pltpu.sync_copy(S_sp.at[pl.ds(base, RPT)],
                        S_out.at[c * 2 + p, pl.ds(base, RPT)])
        plsc.subcore_barrier()


def _deg_body(dst_hbm, deg_out, dstv, onesv, zbd, deg_sp, sem1):
    c = lax.axis_index("c")
    s = lax.axis_index("s")

    def _zinit(e, _):
        zbd[e, :] = jnp.zeros((GW,), jnp.float32)
        return 0
    lax.fori_loop(0, ZR, _zinit, 0)

    def _oinit(e, _):
        onesv[e, :] = jnp.ones((GW,), jnp.float32)
        return 0
    lax.fori_loop(0, CH, _oinit, 0)

    base = s * RPT
    for j in range(RPT // ZR):
        pltpu.sync_copy(zbd, deg_sp.at[pl.ds(base + j * ZR, ZR)])
    plsc.subcore_barrier()

    def _chunk(i, _):
        k = i * NTILE + s

        @pl.when(k < HCHUNK)
        def _():
            off = (c * HCHUNK + k) * CH
            pltpu.sync_copy(dst_hbm.at[pl.ds(off, CH)], dstv)
            pltpu.sync_copy(onesv, deg_sp.at[dstv], add=True)
        return 0
    lax.fori_loop(0, HITERS, _chunk, 0)
    plsc.subcore_barrier()

    pltpu.sync_copy(deg_sp.at[pl.ds(base, RPT)],
                    deg_out.at[c, pl.ds(base, RPT)])
    plsc.subcore_barrier()


def _make_edge_kernel():
    mesh = plsc.VectorSubcoreMesh(core_axis_name="c", subcore_axis_name="s")
    scratch = [
        pltpu.VMEM((BLKE,), jnp.int32),
        pltpu.VMEM((BLKE,), jnp.int32),
        pltpu.VMEM((BLKE,), jnp.int32),
        pltpu.VMEM((CH, GW), jnp.float32),
        pltpu.VMEM((CH, GW), jnp.float32),
        pltpu.VMEM((CH, GW), jnp.float32),
        pltpu.VMEM((CH, GW), jnp.float32),
        pltpu.VMEM((ZR, GW), jnp.float32),
        pltpu.VMEM_SHARED((NN, GW), jnp.float32),
        pltpu.SemaphoreType.DMA,
        pltpu.SemaphoreType.DMA,
        pltpu.SemaphoreType.DMA,
        pltpu.SemaphoreType.DMA,
    ]
    return pl.kernel(_edge_body,
                     out_type=jax.ShapeDtypeStruct((NGRP, NN, GW),
                                                   jnp.float32),
                     mesh=mesh, scratch_types=scratch,
                     compiler_params=pltpu.CompilerParams(
                         use_tc_tiling_on_sc=False))


def _make_deg_kernel():
    mesh = plsc.VectorSubcoreMesh(core_axis_name="c", subcore_axis_name="s")
    scratch = [
        pltpu.VMEM((CH,), jnp.int32),
        pltpu.VMEM((CH, GW), jnp.float32),
        pltpu.VMEM((ZR, GW), jnp.float32),
        pltpu.VMEM_SHARED((NN, GW), jnp.float32),
        pltpu.SemaphoreType.DMA,
    ]
    return pl.kernel(_deg_body,
                     out_type=jax.ShapeDtypeStruct((NSC, NN, GW),
                                                   jnp.float32),
                     mesh=mesh, scratch_types=scratch,
                     compiler_params=pltpu.CompilerParams(
                         use_tc_tiling_on_sc=False))


_edge_kernel = _make_edge_kernel()
_deg_kernel = _make_deg_kernel()


def _split_groups(res, ref):
    for g in range(NGRP):
        ref[g] = res[:, g * GW:(g + 1) * GW]


def _pre_body(x_ref, wt_ref, wb_ref, b1_ref, a_ref, bo_ref):
    xb = x_ref[...]
    _split_groups(_dot(xb, wt_ref[...]) + b1_ref[...], a_ref)
    _split_groups(_dot(xb, wb_ref[...]), bo_ref)


def _tc_pre(x, wt, wb, b1row):
    return pl.pallas_call(
        _pre_body,
        grid=(NROWB,),
        in_specs=[
            pl.BlockSpec((RB, FF), lambda i: (i, 0)),
            pl.BlockSpec((FF, HH), lambda i: (0, 0)),
            pl.BlockSpec((FF, HH), lambda i: (0, 0)),
            pl.BlockSpec((1, HH), lambda i: (0, 0)),
        ],
        out_specs=[pl.BlockSpec((NGRP, RB, GW), lambda i: (0, i, 0))] * 2,
        out_shape=[jax.ShapeDtypeStruct((NGRP, NN, GW), jnp.float32)] * 2,
    )(x, wt, wb, b1row)


def _cat_groups(s_ref):
    return jnp.concatenate([s_ref[g] for g in range(NGRP)], axis=1)


def _mid_body(s_ref, deg_ref, w2_ref, b2_ref, wt_ref, wb_ref, b1_ref,
              a_ref, bo_ref):
    dcol = deg_ref[0][:, 0:1] + deg_ref[1][:, 0:1]
    h = jnp.maximum(_dot(_cat_groups(s_ref), w2_ref[...])
                    + dcol * b2_ref[...], 0.0)
    _split_groups(_dot(h, wt_ref[...]) + b1_ref[...], a_ref)
    _split_groups(_dot(h, wb_ref[...]), bo_ref)


def _tc_mid(S1, deg2, w2, b2row, wt, wb, b1row):
    return pl.pallas_call(
        _mid_body,
        grid=(NROWB,),
        in_specs=[
            pl.BlockSpec((NGRP, RB, GW), lambda i: (0, i, 0)),
            pl.BlockSpec((NSC, RB, GW), lambda i: (0, i, 0)),
            pl.BlockSpec((HH, HH), lambda i: (0, 0)),
            pl.BlockSpec((1, HH), lambda i: (0, 0)),
            pl.BlockSpec((HH, HH), lambda i: (0, 0)),
            pl.BlockSpec((HH, HH), lambda i: (0, 0)),
            pl.BlockSpec((1, HH), lambda i: (0, 0)),
        ],
        out_specs=[pl.BlockSpec((NGRP, RB, GW), lambda i: (0, i, 0))] * 2,
        out_shape=[jax.ShapeDtypeStruct((NGRP, NN, GW), jnp.float32)] * 2,
    )(S1, deg2, w2, b2row, wt, wb, b1row)


def _post_body(s_ref, deg_ref, batch_ref, w2_ref, b2_ref,
               wm1_ref, bm1_ref, wm2_ref, bm2_ref,
               out_ref, sums, cnts):
    i = pl.program_id(0)

    @pl.when(i == 0)
    def _():
        sums[...] = jnp.zeros_like(sums)
        cnts[...] = jnp.zeros_like(cnts)

    dcol = deg_ref[0][:, 0:1] + deg_ref[1][:, 0:1]
    h2 = jnp.maximum(_dot(_cat_groups(s_ref), w2_ref[...])
                     + dcol * b2_ref[...], 0.0)
    bb = batch_ref[0]
    gids = lax.broadcasted_iota(jnp.int32, (GG, RB), 0)
    oh = (bb == gids).astype(jnp.float32)
    sums[...] += lax.dot_general(oh, h2, (((1,), (0,)), ((), ())),
                                 precision=_HI,
                                 preferred_element_type=jnp.float32)
    cnts[...] += lax.dot_general(oh, jnp.ones((RB, HH), jnp.float32),
                                 (((1,), (0,)), ((), ())),
                                 precision=_HI,
                                 preferred_element_type=jnp.float32)

    @pl.when(i == NROWB - 1)
    def _():
        pooled = sums[...] / jnp.maximum(cnts[...], 1.0)
        lat = jnp.maximum(_dot(pooled, wm1_ref[...]) + bm1_ref[...], 0.0)
        out_ref[...] = _dot(lat, wm2_ref[...]) + bm2_ref[...]


def _tc_post(S2, deg2, batch3, w2, b2row, wm1, bm1row, wm2, bm2row):
    return pl.pallas_call(
        _post_body,
        grid=(NROWB,),
        in_specs=[
            pl.BlockSpec((NGRP, RB, GW), lambda i: (0, i, 0)),
            pl.BlockSpec((NSC, RB, GW), lambda i: (0, i, 0)),
            pl.BlockSpec((1, 1, RB), lambda i: (i, 0, 0)),
            pl.BlockSpec((HH, HH), lambda i: (0, 0)),
            pl.BlockSpec((1, HH), lambda i: (0, 0)),
            pl.BlockSpec((HH, HH), lambda i: (0, 0)),
            pl.BlockSpec((1, HH), lambda i: (0, 0)),
            pl.BlockSpec((HH, 2 * HH), lambda i: (0, 0)),
            pl.BlockSpec((1, 2 * HH), lambda i: (0, 0)),
        ],
        out_specs=pl.BlockSpec((GG, 2 * HH), lambda i: (0, 0)),
        out_shape=jax.ShapeDtypeStruct((GG, 2 * HH), jnp.float32),
        scratch_shapes=[
            pltpu.VMEM((GG, HH), jnp.float32),
            pltpu.VMEM((GG, HH), jnp.float32),
        ],
    )(S2, deg2, batch3, w2, b2row, wm1, bm1row, wm2, bm2row)


def kernel(x, W1a, b1a, W2a, b2a, W1b, b1b, W2b, b2b, Wm1, bm1, Wm2, bm2,
           edge_index, batch):
    src = edge_index[0].astype(jnp.int32)
    dst = edge_index[1].astype(jnp.int32)

    A1, B1 = _tc_pre(x, W1a[:FF], W1a[FF:], b1a[None, :])
    deg2 = _deg_kernel(dst)
    S1 = _edge_kernel(A1.reshape(NGRP * NN, GW), B1.reshape(NGRP * NN, GW),
                      src, dst)

    A2, B2 = _tc_mid(S1, deg2, W2a, b2a[None, :], W1b[:HH], W1b[HH:],
                     b1b[None, :])
    S2 = _edge_kernel(A2.reshape(NGRP * NN, GW), B2.reshape(NGRP * NN, GW),
                      src, dst)

    lat = _tc_post(S2, deg2, batch.astype(jnp.int32).reshape(NROWB, 1, RB),
                   W2b, b2b[None, :], Wm1, bm1[None, :], Wm2, bm2[None, :])
    return lat.reshape(-1, 8, 16)


# speedup vs baseline: 6.0051x; 1.0668x over previous
import functools

import jax
import jax.numpy as jnp
from jax import lax
from jax.experimental import pallas as pl
from jax.experimental.pallas import tpu as pltpu
from jax.experimental.pallas import tpu_sc as plsc

NN = 100000
EE = 1600000
FF = 16
HH = 64
GG = 16

NSC = 2
NTILE = 16
NGRP = 4
GW = 16
RPT = NN // NTILE
CH = 80
EPT = EE // NTILE
BLKE = 2000
SUBC = BLKE // CH
NBLK = EPT // BLKE
NCHUNK = EE // 128
HCHUNK = NCHUNK // NSC
HITERS = (HCHUNK + NTILE - 1) // NTILE
ZR = 250

RB = 2000
NROWB = NN // RB

_HI = jax.lax.Precision.HIGHEST


def _dot(a, b):
    return jnp.dot(a, b, precision=_HI, preferred_element_type=jnp.float32)


def _edge_body(A_hbm, B_hbm, src_hbm, dst_hbm, dep_hbm, S_out,
               srcv, dstv, gidxv, av0, bv0, av1, bv1, zb, S_sp,
               sa0, sb0, sa1, sb1):
    del dep_hbm
    c = lax.axis_index("c")
    s = lax.axis_index("s")

    def _zinit(e, _):
        zb[e, :] = jnp.zeros((GW,), jnp.float32)
        return 0
    lax.fori_loop(0, ZR, _zinit, 0)

    base = s * RPT
    ebase = s * EPT
    avs = [av0, av1]
    bvs = [bv0, bv1]
    sas = [sa0, sa1]
    sbs = [sb0, sb1]

    for p in range(2):
        g = c * 2 + p

        for j in range(RPT // ZR):
            pltpu.sync_copy(zb, S_sp.at[pl.ds(base + j * ZR, ZR)])
        plsc.subcore_barrier()

        def _block(i, _):
            boff = ebase + i * BLKE
            pltpu.sync_copy(src_hbm.at[pl.ds(boff, BLKE)], srcv)
            pltpu.sync_copy(dst_hbm.at[pl.ds(boff, BLKE)], dstv)

            def _remap(e, _):
                sl = pl.ds(e * 16, 16)
                gidxv[sl] = dstv[sl] * 4 + g
                srcv[sl] = srcv[sl] * 4 + g
                return 0
            lax.fori_loop(0, BLKE // 16, _remap, 0)

            def _issue(j):
                sl = pl.ds(j * CH, CH)
                t = j % 2
                return (pltpu.async_copy(A_hbm.at[gidxv.at[sl]], avs[t],
                                         sas[t]),
                        pltpu.async_copy(B_hbm.at[srcv.at[sl]], bvs[t],
                                         sbs[t]))

            cur = _issue(0)
            for j in range(SUBC):
                t = j % 2
                nxt = _issue(j + 1) if j + 1 < SUBC else None
                cur[0].wait()
                cur[1].wait()
                av, bv = avs[t], bvs[t]

                def _relu(e, _):
                    rb = e * 8
                    for u in range(8):
                        av[rb + u, :] = jnp.maximum(
                            av[rb + u, :] + bv[rb + u, :], 0.0)
                    return 0
                lax.fori_loop(0, CH // 8, _relu, 0)

                pltpu.sync_copy(av, S_sp.at[dstv.at[pl.ds(j * CH, CH)]],
                                add=True)
                cur = nxt
            return 0
        lax.fori_loop(0, NBLK, _block, 0)
        plsc.subcore_barrier()

        pltpu.sync_copy(S_sp.at[pl.ds(base, RPT)],
                        S_out.at[pl.ds(base, RPT), g])
        plsc.subcore_barrier()


def _deg_body(dst_hbm, deg_out, dstv, onesv, zbd, deg_sp, sem1):
    c = lax.axis_index("c")
    s = lax.axis_index("s")

    def _zinit(e, _):
        zbd[e, :] = jnp.zeros((GW,), jnp.float32)
        return 0
    lax.fori_loop(0, ZR, _zinit, 0)

    def _oinit(e, _):
        onesv[e, :] = jnp.ones((GW,), jnp.float32)
        return 0
    lax.fori_loop(0, 128, _oinit, 0)

    base = s * RPT
    for j in range(RPT // ZR):
        pltpu.sync_copy(zbd, deg_sp.at[pl.ds(base + j * ZR, ZR)])
    plsc.subcore_barrier()

    def _chunk(i, _):
        k = i * NTILE + s

        @pl.when(k < HCHUNK)
        def _():
            off = (c * HCHUNK + k) * 128
            pltpu.sync_copy(dst_hbm.at[pl.ds(off, 128)], dstv)
            pltpu.sync_copy(onesv, deg_sp.at[dstv], add=True)
        return 0
    lax.fori_loop(0, HITERS, _chunk, 0)
    plsc.subcore_barrier()

    pltpu.sync_copy(deg_sp.at[pl.ds(base, RPT)],
                    deg_out.at[pl.ds(base, RPT), c])
    plsc.subcore_barrier()


def _make_edge_kernel():
    mesh = plsc.VectorSubcoreMesh(core_axis_name="c", subcore_axis_name="s")
    scratch = [
        pltpu.VMEM((BLKE,), jnp.int32),
        pltpu.VMEM((BLKE,), jnp.int32),
        pltpu.VMEM((BLKE,), jnp.int32),
        pltpu.VMEM((CH, GW), jnp.float32),
        pltpu.VMEM((CH, GW), jnp.float32),
        pltpu.VMEM((CH, GW), jnp.float32),
        pltpu.VMEM((CH, GW), jnp.float32),
        pltpu.VMEM((ZR, GW), jnp.float32),
        pltpu.VMEM_SHARED((NN, GW), jnp.float32),
        pltpu.SemaphoreType.DMA,
        pltpu.SemaphoreType.DMA,
        pltpu.SemaphoreType.DMA,
        pltpu.SemaphoreType.DMA,
    ]
    return pl.kernel(_edge_body,
                     out_type=jax.ShapeDtypeStruct((NN, NGRP, GW),
                                                   jnp.float32),
                     mesh=mesh, scratch_types=scratch,
                     compiler_params=pltpu.CompilerParams(
                         use_tc_tiling_on_sc=False))


def _make_deg_kernel():
    mesh = plsc.VectorSubcoreMesh(core_axis_name="c", subcore_axis_name="s")
    scratch = [
        pltpu.VMEM((128,), jnp.int32),
        pltpu.VMEM((128, GW), jnp.float32),
        pltpu.VMEM((ZR, GW), jnp.float32),
        pltpu.VMEM_SHARED((NN, GW), jnp.float32),
        pltpu.SemaphoreType.DMA,
    ]
    return pl.kernel(_deg_body,
                     out_type=jax.ShapeDtypeStruct((NN, NSC, GW),
                                                   jnp.float32),
                     mesh=mesh, scratch_types=scratch,
                     compiler_params=pltpu.CompilerParams(
                         use_tc_tiling_on_sc=False))


_edge_kernel = _make_edge_kernel()
_deg_kernel = _make_deg_kernel()


def _pre_body(x_ref, wt_ref, wb_ref, b1_ref, a_ref, bo_ref):
    xb = x_ref[...]
    a_ref[...] = _dot(xb, wt_ref[...]) + b1_ref[...]
    bo_ref[...] = _dot(xb, wb_ref[...])


def _tc_pre(x, wt, wb, b1row):
    return pl.pallas_call(
        _pre_body,
        grid=(NROWB,),
        in_specs=[
            pl.BlockSpec((RB, FF), lambda i: (i, 0)),
            pl.BlockSpec((FF, HH), lambda i: (0, 0)),
            pl.BlockSpec((FF, HH), lambda i: (0, 0)),
            pl.BlockSpec((1, HH), lambda i: (0, 0)),
        ],
        out_specs=[pl.BlockSpec((RB, HH), lambda i: (i, 0))] * 2,
        out_shape=[jax.ShapeDtypeStruct((NN, HH), jnp.float32)] * 2,
    )(x, wt, wb, b1row)


def _dcol(deg_ref):
    return deg_ref[:, 0:1] + deg_ref[:, GW:GW + 1]


def _mid_body(s_ref, deg_ref, w2_ref, b2_ref, wt_ref, wb_ref, b1_ref,
              a_ref, bo_ref):
    h = jnp.maximum(_dot(s_ref[...], w2_ref[...])
                    + _dcol(deg_ref) * b2_ref[...], 0.0)
    a_ref[...] = _dot(h, wt_ref[...]) + b1_ref[...]
    bo_ref[...] = _dot(h, wb_ref[...])


def _tc_mid(S1, deg2, w2, b2row, wt, wb, b1row):
    return pl.pallas_call(
        _mid_body,
        grid=(NROWB,),
        in_specs=[
            pl.BlockSpec((RB, HH), lambda i: (i, 0)),
            pl.BlockSpec((RB, NSC * GW), lambda i: (i, 0)),
            pl.BlockSpec((HH, HH), lambda i: (0, 0)),
            pl.BlockSpec((1, HH), lambda i: (0, 0)),
            pl.BlockSpec((HH, HH), lambda i: (0, 0)),
            pl.BlockSpec((HH, HH), lambda i: (0, 0)),
            pl.BlockSpec((1, HH), lambda i: (0, 0)),
        ],
        out_specs=[pl.BlockSpec((RB, HH), lambda i: (i, 0))] * 2,
        out_shape=[jax.ShapeDtypeStruct((NN, HH), jnp.float32)] * 2,
    )(S1, deg2, w2, b2row, wt, wb, b1row)


def _post_body(s_ref, deg_ref, batch_ref, w2_ref, b2_ref,
               wm1_ref, bm1_ref, wm2_ref, bm2_ref,
               out_ref, sums, cnts):
    i = pl.program_id(0)

    @pl.when(i == 0)
    def _():
        sums[...] = jnp.zeros_like(sums)
        cnts[...] = jnp.zeros_like(cnts)

    h2 = jnp.maximum(_dot(s_ref[...], w2_ref[...])
                     + _dcol(deg_ref) * b2_ref[...], 0.0)
    bb = batch_ref[0]
    gids = lax.broadcasted_iota(jnp.int32, (GG, RB), 0)
    oh = (bb == gids).astype(jnp.float32)
    sums[...] += lax.dot_general(oh, h2, (((1,), (0,)), ((), ())),
                                 precision=_HI,
                                 preferred_element_type=jnp.float32)
    cnts[...] += lax.dot_general(oh, jnp.ones((RB, HH), jnp.float32),
                                 (((1,), (0,)), ((), ())),
                                 precision=_HI,
                                 preferred_element_type=jnp.float32)

    @pl.when(i == NROWB - 1)
    def _():
        pooled = sums[...] / jnp.maximum(cnts[...], 1.0)
        lat = jnp.maximum(_dot(pooled, wm1_ref[...]) + bm1_ref[...], 0.0)
        out_ref[...] = _dot(lat, wm2_ref[...]) + bm2_ref[...]


def _tc_post(S2, deg2, batch3, w2, b2row, wm1, bm1row, wm2, bm2row):
    return pl.pallas_call(
        _post_body,
        grid=(NROWB,),
        in_specs=[
            pl.BlockSpec((RB, HH), lambda i: (i, 0)),
            pl.BlockSpec((RB, NSC * GW), lambda i: (i, 0)),
            pl.BlockSpec((1, 1, RB), lambda i: (i, 0, 0)),
            pl.BlockSpec((HH, HH), lambda i: (0, 0)),
            pl.BlockSpec((1, HH), lambda i: (0, 0)),
            pl.BlockSpec((HH, HH), lambda i: (0, 0)),
            pl.BlockSpec((1, HH), lambda i: (0, 0)),
            pl.BlockSpec((HH, 2 * HH), lambda i: (0, 0)),
            pl.BlockSpec((1, 2 * HH), lambda i: (0, 0)),
        ],
        out_specs=pl.BlockSpec((GG, 2 * HH), lambda i: (0, 0)),
        out_shape=jax.ShapeDtypeStruct((GG, 2 * HH), jnp.float32),
        scratch_shapes=[
            pltpu.VMEM((GG, HH), jnp.float32),
            pltpu.VMEM((GG, HH), jnp.float32),
        ],
    )(S2, deg2, batch3, w2, b2row, wm1, bm1row, wm2, bm2row)


def kernel(x, W1a, b1a, W2a, b2a, W1b, b1b, W2b, b2b, Wm1, bm1, Wm2, bm2,
           edge_index, batch):
    src = edge_index[0].astype(jnp.int32)
    dst = edge_index[1].astype(jnp.int32)

    A1, B1 = _tc_pre(x, W1a[:FF], W1a[FF:], b1a[None, :])
    deg2 = _deg_kernel(dst)
    degr = deg2.reshape(NN, NSC * GW)
    S1 = _edge_kernel(A1.reshape(NGRP * NN, GW), B1.reshape(NGRP * NN, GW),
                      src, dst, deg2).reshape(NN, HH)

    A2, B2 = _tc_mid(S1, degr, W2a, b2a[None, :], W1b[:HH], W1b[HH:],
                     b1b[None, :])
    S2 = _edge_kernel(A2.reshape(NGRP * NN, GW), B2.reshape(NGRP * NN, GW),
                      src, dst, deg2).reshape(NN, HH)

    lat = _tc_post(S2, degr, batch.astype(jnp.int32).reshape(NROWB, 1, RB),
                   W2b, b2b[None, :], Wm1, bm1[None, :], Wm2, bm2[None, :])
    return lat.reshape(-1, 8, 16)


# speedup vs baseline: 6.2100x; 1.0341x over previous
import functools

import jax
import jax.numpy as jnp
from jax import lax
from jax.experimental import pallas as pl
from jax.experimental.pallas import tpu as pltpu
from jax.experimental.pallas import tpu_sc as plsc

NN = 100000
EE = 1600000
FF = 16
HH = 64
GG = 16

NSC = 2
NTILE = 16
NGRP = 4
GW = 16
RPT = NN // NTILE
CH = 80
EPT = EE // NTILE
BLKE = 4000
SUBC = BLKE // CH
NBLK = EPT // BLKE
NCHUNK = EE // 128
HCHUNK = NCHUNK // NSC
HITERS = (HCHUNK + NTILE - 1) // NTILE
ZR = 250

RB = 2000
NROWB = NN // RB

_HI = jax.lax.Precision.HIGHEST


def _dot(a, b):
    return jnp.dot(a, b, precision=_HI, preferred_element_type=jnp.float32)


def _edge_body(A_hbm, B_hbm, src_hbm, dst_hbm, dep_hbm, S_out,
               srcv, dstv, gidxv, av0, bv0, av1, bv1, zb, S_sp,
               sa0, sb0, sa1, sb1):
    del dep_hbm
    c = lax.axis_index("c")
    s = lax.axis_index("s")

    def _zinit(e, _):
        zb[e, :] = jnp.zeros((GW,), jnp.float32)
        return 0
    lax.fori_loop(0, ZR, _zinit, 0)

    base = s * RPT
    ebase = s * EPT
    avs = [av0, av1]
    bvs = [bv0, bv1]
    sas = [sa0, sa1]
    sbs = [sb0, sb1]

    for p in range(2):
        g = c * 2 + p

        for j in range(RPT // ZR):
            pltpu.sync_copy(zb, S_sp.at[pl.ds(base + j * ZR, ZR)])
        plsc.subcore_barrier()

        def _block(i, _):
            boff = ebase + i * BLKE
            pltpu.sync_copy(src_hbm.at[pl.ds(boff, BLKE)], srcv)
            pltpu.sync_copy(dst_hbm.at[pl.ds(boff, BLKE)], dstv)

            def _remap(e, _):
                sl = pl.ds(e * 16, 16)
                gidxv[sl] = dstv[sl] * 4 + g
                srcv[sl] = srcv[sl] * 4 + g
                return 0
            lax.fori_loop(0, BLKE // 16, _remap, 0)

            def _issue(j):
                sl = pl.ds(j * CH, CH)
                t = j % 2
                return (pltpu.async_copy(A_hbm.at[gidxv.at[sl]], avs[t],
                                         sas[t]),
                        pltpu.async_copy(B_hbm.at[srcv.at[sl]], bvs[t],
                                         sbs[t]))

            cur = _issue(0)
            for j in range(SUBC):
                t = j % 2
                nxt = _issue(j + 1) if j + 1 < SUBC else None
                cur[0].wait()
                cur[1].wait()
                av, bv = avs[t], bvs[t]

                def _relu(e, _):
                    rb = e * 8
                    for u in range(8):
                        av[rb + u, :] = jnp.maximum(
                            av[rb + u, :] + bv[rb + u, :], 0.0)
                    return 0
                lax.fori_loop(0, CH // 8, _relu, 0)

                pltpu.sync_copy(av, S_sp.at[dstv.at[pl.ds(j * CH, CH)]],
                                add=True)
                cur = nxt
            return 0
        lax.fori_loop(0, NBLK, _block, 0)
        plsc.subcore_barrier()

        pltpu.sync_copy(S_sp.at[pl.ds(base, RPT)],
                        S_out.at[pl.ds(base, RPT), g])
        plsc.subcore_barrier()


def _deg_body(dst_hbm, deg_out, dstv, onesv, zbd, deg_sp, sem1):
    c = lax.axis_index("c")
    s = lax.axis_index("s")

    def _zinit(e, _):
        zbd[e, :] = jnp.zeros((GW,), jnp.float32)
        return 0
    lax.fori_loop(0, ZR, _zinit, 0)

    def _oinit(e, _):
        onesv[e, :] = jnp.ones((GW,), jnp.float32)
        return 0
    lax.fori_loop(0, 128, _oinit, 0)

    base = s * RPT
    for j in range(RPT // ZR):
        pltpu.sync_copy(zbd, deg_sp.at[pl.ds(base + j * ZR, ZR)])
    plsc.subcore_barrier()

    def _chunk(i, _):
        k = i * NTILE + s

        @pl.when(k < HCHUNK)
        def _():
            off = (c * HCHUNK + k) * 128
            pltpu.sync_copy(dst_hbm.at[pl.ds(off, 128)], dstv)
            pltpu.sync_copy(onesv, deg_sp.at[dstv], add=True)
        return 0
    lax.fori_loop(0, HITERS, _chunk, 0)
    plsc.subcore_barrier()

    pltpu.sync_copy(deg_sp.at[pl.ds(base, RPT)],
                    deg_out.at[pl.ds(base, RPT), c])
    plsc.subcore_barrier()


def _make_edge_kernel():
    mesh = plsc.VectorSubcoreMesh(core_axis_name="c", subcore_axis_name="s")
    scratch = [
        pltpu.VMEM((BLKE,), jnp.int32),
        pltpu.VMEM((BLKE,), jnp.int32),
        pltpu.VMEM((BLKE,), jnp.int32),
        pltpu.VMEM((CH, GW), jnp.float32),
        pltpu.VMEM((CH, GW), jnp.float32),
        pltpu.VMEM((CH, GW), jnp.float32),
        pltpu.VMEM((CH, GW), jnp.float32),
        pltpu.VMEM((ZR, GW), jnp.float32),
        pltpu.VMEM_SHARED((NN, GW), jnp.float32),
        pltpu.SemaphoreType.DMA,
        pltpu.SemaphoreType.DMA,
        pltpu.SemaphoreType.DMA,
        pltpu.SemaphoreType.DMA,
    ]
    return pl.kernel(_edge_body,
                     out_type=jax.ShapeDtypeStruct((NN, NGRP, GW),
                                                   jnp.float32),
                     mesh=mesh, scratch_types=scratch,
                     compiler_params=pltpu.CompilerParams(
                         use_tc_tiling_on_sc=False))


def _make_deg_kernel():
    mesh = plsc.VectorSubcoreMesh(core_axis_name="c", subcore_axis_name="s")
    scratch = [
        pltpu.VMEM((128,), jnp.int32),
        pltpu.VMEM((128, GW), jnp.float32),
        pltpu.VMEM((ZR, GW), jnp.float32),
        pltpu.VMEM_SHARED((NN, GW), jnp.float32),
        pltpu.SemaphoreType.DMA,
    ]
    return pl.kernel(_deg_body,
                     out_type=jax.ShapeDtypeStruct((NN, NSC, GW),
                                                   jnp.float32),
                     mesh=mesh, scratch_types=scratch,
                     compiler_params=pltpu.CompilerParams(
                         use_tc_tiling_on_sc=False))


_edge_kernel = _make_edge_kernel()
_deg_kernel = _make_deg_kernel()


def _pre_body(x_ref, wt_ref, wb_ref, b1_ref, a_ref, bo_ref):
    xb = x_ref[...]
    a_ref[...] = _dot(xb, wt_ref[...]) + b1_ref[...]
    bo_ref[...] = _dot(xb, wb_ref[...])


def _tc_pre(x, wt, wb, b1row):
    return pl.pallas_call(
        _pre_body,
        grid=(NROWB,),
        in_specs=[
            pl.BlockSpec((RB, FF), lambda i: (i, 0)),
            pl.BlockSpec((FF, HH), lambda i: (0, 0)),
            pl.BlockSpec((FF, HH), lambda i: (0, 0)),
            pl.BlockSpec((1, HH), lambda i: (0, 0)),
        ],
        out_specs=[pl.BlockSpec((RB, HH), lambda i: (i, 0))] * 2,
        out_shape=[jax.ShapeDtypeStruct((NN, HH), jnp.float32)] * 2,
    )(x, wt, wb, b1row)


def _dcol(deg_ref):
    return deg_ref[:, 0:1] + deg_ref[:, GW:GW + 1]


def _mid_body(s_ref, deg_ref, w2_ref, b2_ref, wt_ref, wb_ref, b1_ref,
              a_ref, bo_ref):
    h = jnp.maximum(_dot(s_ref[...], w2_ref[...])
                    + _dcol(deg_ref) * b2_ref[...], 0.0)
    a_ref[...] = _dot(h, wt_ref[...]) + b1_ref[...]
    bo_ref[...] = _dot(h, wb_ref[...])


def _tc_mid(S1, deg2, w2, b2row, wt, wb, b1row):
    return pl.pallas_call(
        _mid_body,
        grid=(NROWB,),
        in_specs=[
            pl.BlockSpec((RB, HH), lambda i: (i, 0)),
            pl.BlockSpec((RB, NSC * GW), lambda i: (i, 0)),
            pl.BlockSpec((HH, HH), lambda i: (0, 0)),
            pl.BlockSpec((1, HH), lambda i: (0, 0)),
            pl.BlockSpec((HH, HH), lambda i: (0, 0)),
            pl.BlockSpec((HH, HH), lambda i: (0, 0)),
            pl.BlockSpec((1, HH), lambda i: (0, 0)),
        ],
        out_specs=[pl.BlockSpec((RB, HH), lambda i: (i, 0))] * 2,
        out_shape=[jax.ShapeDtypeStruct((NN, HH), jnp.float32)] * 2,
    )(S1, deg2, w2, b2row, wt, wb, b1row)


def _post_body(s_ref, deg_ref, batch_ref, w2_ref, b2_ref,
               wm1_ref, bm1_ref, wm2_ref, bm2_ref,
               out_ref, sums, cnts):
    i = pl.program_id(0)

    @pl.when(i == 0)
    def _():
        sums[...] = jnp.zeros_like(sums)
        cnts[...] = jnp.zeros_like(cnts)

    h2 = jnp.maximum(_dot(s_ref[...], w2_ref[...])
                     + _dcol(deg_ref) * b2_ref[...], 0.0)
    bb = batch_ref[0]
    gids = lax.broadcasted_iota(jnp.int32, (GG, RB), 0)
    oh = (bb == gids).astype(jnp.float32)
    sums[...] += lax.dot_general(oh, h2, (((1,), (0,)), ((), ())),
                                 precision=_HI,
                                 preferred_element_type=jnp.float32)
    cnts[...] += lax.dot_general(oh, jnp.ones((RB, HH), jnp.float32),
                                 (((1,), (0,)), ((), ())),
                                 precision=_HI,
                                 preferred_element_type=jnp.float32)

    @pl.when(i == NROWB - 1)
    def _():
        pooled = sums[...] / jnp.maximum(cnts[...], 1.0)
        lat = jnp.maximum(_dot(pooled, wm1_ref[...]) + bm1_ref[...], 0.0)
        out_ref[...] = _dot(lat, wm2_ref[...]) + bm2_ref[...]


def _tc_post(S2, deg2, batch3, w2, b2row, wm1, bm1row, wm2, bm2row):
    return pl.pallas_call(
        _post_body,
        grid=(NROWB,),
        in_specs=[
            pl.BlockSpec((RB, HH), lambda i: (i, 0)),
            pl.BlockSpec((RB, NSC * GW), lambda i: (i, 0)),
            pl.BlockSpec((1, 1, RB), lambda i: (i, 0, 0)),
            pl.BlockSpec((HH, HH), lambda i: (0, 0)),
            pl.BlockSpec((1, HH), lambda i: (0, 0)),
            pl.BlockSpec((HH, HH), lambda i: (0, 0)),
            pl.BlockSpec((1, HH), lambda i: (0, 0)),
            pl.BlockSpec((HH, 2 * HH), lambda i: (0, 0)),
            pl.BlockSpec((1, 2 * HH), lambda i: (0, 0)),
        ],
        out_specs=pl.BlockSpec((GG, 2 * HH), lambda i: (0, 0)),
        out_shape=jax.ShapeDtypeStruct((GG, 2 * HH), jnp.float32),
        scratch_shapes=[
            pltpu.VMEM((GG, HH), jnp.float32),
            pltpu.VMEM((GG, HH), jnp.float32),
        ],
    )(S2, deg2, batch3, w2, b2row, wm1, bm1row, wm2, bm2row)


def kernel(x, W1a, b1a, W2a, b2a, W1b, b1b, W2b, b2b, Wm1, bm1, Wm2, bm2,
           edge_index, batch):
    src = edge_index[0].astype(jnp.int32)
    dst = edge_index[1].astype(jnp.int32)

    A1, B1 = _tc_pre(x, W1a[:FF], W1a[FF:], b1a[None, :])
    deg2 = _deg_kernel(dst)
    degr = deg2.reshape(NN, NSC * GW)
    S1 = _edge_kernel(A1.reshape(NGRP * NN, GW), B1.reshape(NGRP * NN, GW),
                      src, dst, deg2).reshape(NN, HH)

    A2, B2 = _tc_mid(S1, degr, W2a, b2a[None, :], W1b[:HH], W1b[HH:],
                     b1b[None, :])
    S2 = _edge_kernel(A2.reshape(NGRP * NN, GW), B2.reshape(NGRP * NN, GW),
                      src, dst, deg2).reshape(NN, HH)

    lat = _tc_post(S2, degr, batch.astype(jnp.int32).reshape(NROWB, 1, RB),
                   W2b, b2b[None, :], Wm1, bm1[None, :], Wm2, bm2[None, :])
    return lat.reshape(-1, 8, 16)
